# baseline (speedup 1.0000x reference)
import functools

import jax
import jax.numpy as jnp
from jax import lax
from jax.experimental import pallas as pl
from jax.experimental.pallas import tpu as pltpu
from jax.experimental.pallas import tpu_sc as plsc


@functools.lru_cache(maxsize=None)
def _make_gather(B, V, D):
    info = plsc.get_sparse_core_info()
    NC, NS, L = info.num_cores, info.num_subcores, info.num_lanes
    NW = NC * NS
    assert B % NW == 0 and D == 64 and L == 16
    b_per_w = B // NW
    C = 128
    assert b_per_w % (2 * C) == 0
    n_chunks = b_per_w // C

    mesh = plsc.VectorSubcoreMesh(core_axis_name="c", subcore_axis_name="s")

    @functools.partial(
        pl.kernel,
        mesh=mesh,
        out_type=jax.ShapeDtypeStruct((B, D), jnp.float32),
        scratch_types=[
            pltpu.VMEM((n_chunks, C), jnp.int32),
            pltpu.VMEM((C, D), jnp.float32),
            pltpu.VMEM((C, D), jnp.float32),
            pltpu.SemaphoreType.DMA,
            pltpu.SemaphoreType.DMA,
            pltpu.SemaphoreType.DMA,
            pltpu.SemaphoreType.DMA,
        ],
    )
    def k(tokens_hbm, table_hbm, out_hbm, idx_v, rows0, rows1,
          sem0, sem1, osem0, osem1):
        wid = lax.axis_index("s") * NC + lax.axis_index("c")
        base = wid * b_per_w
        pltpu.sync_copy(tokens_hbm.at[wid], idx_v)

        def fire(j, buf, sem):
            def group(g, carry):
                v16 = idx_v[j, pl.ds(g * L, L)]
                band = jax.lax.shift_right_logical(v16, 3)
                sub = jax.lax.bitwise_and(v16, 7)
                for l in range(L):
                    pltpu.async_copy(
                        table_hbm.at[pl.ds(band[l], 1), pl.ds(sub[l], 1)],
                        buf.at[pl.ds(g * L + l, 1)].reshape(1, 1, D),
                        sem,
                    )
                return carry

            lax.fori_loop(0, C // L, group, 0)

        def drain(buf, sem):
            pltpu.make_async_copy(
                table_hbm.at[pl.ds(0, C // 8)], buf.reshape(C // 8, 8, D), sem
            ).wait()

        def start_write(j, buf, osem):
            pltpu.async_copy(buf, out_hbm.at[pl.ds(base + j * C, C)], osem)

        def wait_write(j, buf, osem):
            pltpu.make_async_copy(
                buf, out_hbm.at[pl.ds(base + j * C, C)], osem).wait()

        fire(0, rows0, sem0)
        fire(1, rows1, sem1)

        def body(jj, carry):
            j0 = 2 * jj
            drain(rows0, sem0)
            start_write(j0, rows0, osem0)
            drain(rows1, sem1)
            start_write(j0 + 1, rows1, osem1)
            wait_write(j0, rows0, osem0)

            @pl.when(j0 + 2 < n_chunks)
            def _():
                fire(j0 + 2, rows0, sem0)

            wait_write(j0 + 1, rows1, osem1)

            @pl.when(j0 + 3 < n_chunks)
            def _f1():
                fire(j0 + 3, rows1, sem1)

            return carry

        lax.fori_loop(0, n_chunks // 2, body, 0)

    return k, NW, n_chunks, C


def kernel(tokens, table):
    Bb, T = tokens.shape
    V, D = table.shape
    B = Bb * T
    k, NW, n_chunks, C = _make_gather(B, V, D)
    idx = tokens.astype(jnp.int32).reshape(NW, n_chunks, C)
    tbl3 = table.reshape(V // 8, 8, D)
    out = k(idx, tbl3)
    return out.reshape(Bb, T, D)


# speedup vs baseline: 1.0092x; 1.0092x over previous
import functools

import jax
import jax.numpy as jnp
from jax import lax
from jax.experimental import pallas as pl
from jax.experimental.pallas import tpu as pltpu
from jax.experimental.pallas import tpu_sc as plsc


@functools.lru_cache(maxsize=None)
def _make_gather(B, V, D):
    info = plsc.get_sparse_core_info()
    NC, NS, L = info.num_cores, info.num_subcores, info.num_lanes
    NW = NC * NS
    assert B % NW == 0 and D == 64 and L == 16
    b_per_w = B // NW
    C = 128
    assert b_per_w % (2 * C) == 0
    n_chunks = b_per_w // C

    mesh = plsc.VectorSubcoreMesh(core_axis_name="c", subcore_axis_name="s")

    @functools.partial(
        pl.kernel,
        mesh=mesh,
        out_type=jax.ShapeDtypeStruct((B, D), jnp.float32),
        scratch_types=[
            pltpu.VMEM((n_chunks, C), jnp.int32),
            pltpu.VMEM((C, D), jnp.float32),
            pltpu.VMEM((C, D), jnp.float32),
            pltpu.SemaphoreType.DMA,
            pltpu.SemaphoreType.DMA,
        ],
    )
    def k(tokens_hbm, table_hbm, out_hbm, idx_v, rows0, rows1, sem0, sem1):
        wid = lax.axis_index("s") * NC + lax.axis_index("c")
        base = wid * b_per_w
        pltpu.sync_copy(tokens_hbm.at[wid], idx_v)

        def fire(j, buf, sem):
            def group(g, carry):
                v16 = idx_v[j, pl.ds(g * L, L)]
                band = jax.lax.shift_right_logical(v16, 3)
                sub = jax.lax.bitwise_and(v16, 7)
                for l in range(L):
                    pltpu.async_copy(
                        table_hbm.at[pl.ds(band[l], 1), pl.ds(sub[l], 1)],
                        buf.at[pl.ds(g * L + l, 1)].reshape(1, 1, D),
                        sem,
                    )
                return carry

            lax.fori_loop(0, C // L, group, 0)

        def drain(buf, sem):
            pltpu.make_async_copy(
                table_hbm.at[pl.ds(0, C // 8)], buf.reshape(C // 8, 8, D), sem
            ).wait()

        def write_out(j, buf):
            pltpu.sync_copy(buf, out_hbm.at[pl.ds(base + j * C, C)])

        fire(0, rows0, sem0)

        def body(jj, carry):
            j0 = 2 * jj
            fire(j0 + 1, rows1, sem1)
            drain(rows0, sem0)
            write_out(j0, rows0)

            @pl.when(j0 + 2 < n_chunks)
            def _():
                fire(j0 + 2, rows0, sem0)

            drain(rows1, sem1)
            write_out(j0 + 1, rows1)
            return carry

        lax.fori_loop(0, n_chunks // 2, body, 0)

    return k, NW, n_chunks, C


def kernel(tokens, table):
    Bb, T = tokens.shape
    V, D = table.shape
    B = Bb * T
    k, NW, n_chunks, C = _make_gather(B, V, D)
    idx = tokens.astype(jnp.int32).reshape(NW, n_chunks, C)
    tbl3 = table.reshape(V // 8, 8, D)
    out = k(idx, tbl3)
    return out.reshape(Bb, T, D)
